# baseline (speedup 1.0000x reference)
import functools

import jax
import jax.numpy as jnp
import numpy as np
from jax import lax
from jax.experimental import pallas as pl
from jax.experimental.pallas import tpu as pltpu

_NUM_PER_CLS = np.array([5000,4773,4556,4349,4151,3963,3782,3611,3447,3290,3141,2998,2862,2732,2608,2489,2376,2268,2165,2067,1973,1883,1798,1716,1638,1564,1493,1425,1360,1298,1239,1183,1129,1078,1029,982,937,895,854,815,778,743,709,677,646,617,589,562,536,512,489,466,445,425,406,387,370,353,337,321,307,293,280,267,255,243,232,222,212,202,193,184,176,168,160,153,146,139,133,127,121,116,110,105,101,96,92,88,84,80,76,73,70,66,63,60,58,55,52,50], dtype=np.float64)
_MAX_M = 0.5
_S = 30.0
_m = 1.0 / np.sqrt(np.sqrt(_NUM_PER_CLS))
_m = _m * (_MAX_M / np.max(_m))
_M_LIST = jnp.asarray(_m[None, :], dtype=jnp.float32)
_beta = 0.9999
_eff = 1.0 - np.power(_beta, _NUM_PER_CLS)
_w = (1.0 - _beta) / np.array(_eff)
_w = _w / np.sum(_w) * len(_NUM_PER_CLS)
_W_CLS = jnp.asarray(_w[None, :], dtype=jnp.float32)


def _ldam_body(nsteps, x_ref, t_ref, m_ref, w_ref, out_ref, acc_ref):
    i = pl.program_id(0)
    x = x_ref[...]
    t = t_ref[...]
    bm, c = x.shape
    j = lax.broadcasted_iota(jnp.int32, (bm, c), 1)
    onehot = j == t
    m = m_ref[...]
    logits = _S * jnp.where(onehot, x - m, x)
    rowmax = jnp.max(logits, axis=1, keepdims=True)
    e = jnp.exp(logits - rowmax)
    sumexp = jnp.sum(e, axis=1)
    lse = rowmax[:, 0] + jnp.log(sumexp)
    tgt_logit = jnp.sum(jnp.where(onehot, logits, 0.0), axis=1)
    ce = lse - tgt_logit
    w = w_ref[...]
    wt = jnp.sum(jnp.where(onehot, w, jnp.zeros_like(w)), axis=1)
    num = jnp.sum(wt * ce)
    den = jnp.sum(wt)

    @pl.when(i == 0)
    def _():
        acc_ref[0] = num
        acc_ref[1] = den

    @pl.when(i > 0)
    def _():
        acc_ref[0] += num
        acc_ref[1] += den

    @pl.when(i == nsteps - 1)
    def _():
        out_ref[0, 0] = acc_ref[0] / acc_ref[1]


@jax.jit
def kernel(x, target):
    b, c = x.shape
    bm = 8192
    nsteps = b // bm
    t2 = target.reshape(b, 1)
    out = pl.pallas_call(
        functools.partial(_ldam_body, nsteps),
        grid=(nsteps,),
        in_specs=[
            pl.BlockSpec((bm, c), lambda i: (i, 0)),
            pl.BlockSpec((bm, 1), lambda i: (i, 0)),
            pl.BlockSpec((1, c), lambda i: (0, 0)),
            pl.BlockSpec((1, c), lambda i: (0, 0)),
        ],
        out_specs=pl.BlockSpec(memory_space=pltpu.SMEM),
        out_shape=jax.ShapeDtypeStruct((1, 1), jnp.float32),
        scratch_shapes=[pltpu.SMEM((2,), jnp.float32)],
        compiler_params=pltpu.CompilerParams(
            dimension_semantics=("arbitrary",),
        ),
    )(x, t2, _M_LIST, _W_CLS)
    return out[0, 0]


# speedup vs baseline: 1.0320x; 1.0320x over previous
import functools

import jax
import jax.numpy as jnp
import numpy as np
from jax import lax
from jax.experimental import pallas as pl
from jax.experimental.pallas import tpu as pltpu

_NUM_PER_CLS = np.array([5000,4773,4556,4349,4151,3963,3782,3611,3447,3290,3141,2998,2862,2732,2608,2489,2376,2268,2165,2067,1973,1883,1798,1716,1638,1564,1493,1425,1360,1298,1239,1183,1129,1078,1029,982,937,895,854,815,778,743,709,677,646,617,589,562,536,512,489,466,445,425,406,387,370,353,337,321,307,293,280,267,255,243,232,222,212,202,193,184,176,168,160,153,146,139,133,127,121,116,110,105,101,96,92,88,84,80,76,73,70,66,63,60,58,55,52,50], dtype=np.float64)
_MAX_M = 0.5
_S = 30.0
_m = 1.0 / np.sqrt(np.sqrt(_NUM_PER_CLS))
_m = _m * (_MAX_M / np.max(_m))
_M_LIST = jnp.asarray(_m[None, :], dtype=jnp.float32)
_beta = 0.9999
_eff = 1.0 - np.power(_beta, _NUM_PER_CLS)
_w = (1.0 - _beta) / np.array(_eff)
_w = _w / np.sum(_w) * len(_NUM_PER_CLS)
_W_CLS = jnp.asarray(_w[None, :], dtype=jnp.float32)


def _ldam_body(nsteps, x_ref, t_ref, m_ref, w_ref, out_ref, acc_ref):
    i = pl.program_id(0)
    x = x_ref[...]
    t = t_ref[...]
    bm, c = x.shape
    j = lax.broadcasted_iota(jnp.int32, (bm, c), 1)
    onehot = j == t
    m = m_ref[...]
    u = jnp.where(onehot, x - m, x)
    rowmax = jnp.max(u, axis=1, keepdims=True)
    e = jnp.exp(_S * (u - rowmax))
    sumexp = jnp.sum(e, axis=1)
    tgt_u = jnp.sum(jnp.where(onehot, u, 0.0), axis=1)
    ce = _S * (rowmax[:, 0] - tgt_u) + jnp.log(sumexp)
    w = w_ref[...]
    wt = jnp.sum(jnp.where(onehot, w, jnp.zeros_like(w)), axis=1)
    num = jnp.sum(wt * ce)
    den = jnp.sum(wt)

    @pl.when(i == 0)
    def _():
        acc_ref[0] = num
        acc_ref[1] = den

    @pl.when(i > 0)
    def _():
        acc_ref[0] += num
        acc_ref[1] += den

    @pl.when(i == nsteps - 1)
    def _():
        out_ref[0, 0] = acc_ref[0] / acc_ref[1]


@jax.jit
def kernel(x, target):
    b, c = x.shape
    bm = 4096
    nsteps = b // bm
    t2 = target.reshape(b, 1)
    out = pl.pallas_call(
        functools.partial(_ldam_body, nsteps),
        grid=(nsteps,),
        in_specs=[
            pl.BlockSpec((bm, c), lambda i: (i, 0)),
            pl.BlockSpec((bm, 1), lambda i: (i, 0)),
            pl.BlockSpec((1, c), lambda i: (0, 0)),
            pl.BlockSpec((1, c), lambda i: (0, 0)),
        ],
        out_specs=pl.BlockSpec(memory_space=pltpu.SMEM),
        out_shape=jax.ShapeDtypeStruct((1, 1), jnp.float32),
        scratch_shapes=[pltpu.SMEM((2,), jnp.float32)],
        compiler_params=pltpu.CompilerParams(
            dimension_semantics=("arbitrary",),
        ),
    )(x, t2, _M_LIST, _W_CLS)
    return out[0, 0]
